# baseline (speedup 1.0000x reference)
import functools

import jax
import jax.numpy as jnp
from jax import lax
from jax.experimental import pallas as pl
from jax.experimental.pallas import tpu as pltpu, tpu_sc as plsc

EMB_DIM = 128
TIMESTEPS = 1000
BATCH = 16384

_NUM_CORES = 2
_NUM_SUBCORES = 16
_NUM_WORKERS = _NUM_CORES * _NUM_SUBCORES
_B_PER_W = BATCH // _NUM_WORKERS
_N_CHUNKS = 4
_CHUNK = _B_PER_W // _N_CHUNKS


def _build_gather():
    mesh = plsc.VectorSubcoreMesh(core_axis_name="c", subcore_axis_name="s")

    @functools.partial(
        pl.kernel,
        out_type=jax.ShapeDtypeStruct((BATCH, EMB_DIM), jnp.float32),
        mesh=mesh,
        scratch_types=[
            pltpu.VMEM((_B_PER_W,), jnp.int32),
            pltpu.VMEM((_B_PER_W, EMB_DIM), jnp.float32),
            pltpu.VMEM_SHARED((TIMESTEPS, EMB_DIM), jnp.float32),
            pltpu.SemaphoreType.DMA((_N_CHUNKS,)),
            pltpu.SemaphoreType.DMA,
            pltpu.SemaphoreType.DMA,
        ],
    )
    def gather_kernel(table_hbm, idx_hbm, out_hbm, idx_v, rows_v, table_sp, gsems, ssem, isem):
        sid = lax.axis_index("s")
        wid = sid * _NUM_CORES + lax.axis_index("c")
        base = wid * _B_PER_W
        rows_lo = sid * 64
        n_rows = jnp.where(sid == _NUM_SUBCORES - 1, TIMESTEPS - 64 * (_NUM_SUBCORES - 1), 64)
        idx_cp = pltpu.async_copy(idx_hbm.at[pl.ds(base, _B_PER_W)], idx_v, isem)
        stage_cp = pltpu.async_copy(
            table_hbm.at[pl.ds(rows_lo, n_rows)],
            table_sp.at[pl.ds(rows_lo, n_rows)],
            ssem)
        stage_cp.wait()
        idx_cp.wait()
        plsc.subcore_barrier()
        gathers = []
        for c in range(_N_CHUNKS):
            lo = c * _CHUNK
            gathers.append(pltpu.async_copy(
                table_sp.at[idx_v.at[pl.ds(lo, _CHUNK)]],
                rows_v.at[pl.ds(lo, _CHUNK)],
                gsems.at[c]))
        scatters = []
        for c in range(_N_CHUNKS):
            lo = c * _CHUNK
            gathers[c].wait()
            scatters.append(pltpu.async_copy(
                rows_v.at[pl.ds(lo, _CHUNK)],
                out_hbm.at[pl.ds(base + lo, _CHUNK)],
                ssem))
        for s in scatters:
            s.wait()

    return gather_kernel


_gather = _build_gather()


@jax.jit
def kernel(timestep, embedding):
    return _gather(embedding, timestep)


# speedup vs baseline: 1.0060x; 1.0060x over previous
import functools

import jax
import jax.numpy as jnp
from jax import lax
from jax.experimental import pallas as pl
from jax.experimental.pallas import tpu as pltpu, tpu_sc as plsc

EMB_DIM = 128
TIMESTEPS = 1000
BATCH = 16384

_NUM_CORES = 2
_NUM_SUBCORES = 16
_NUM_WORKERS = _NUM_CORES * _NUM_SUBCORES
_B_PER_W = BATCH // _NUM_WORKERS
_N_CHUNKS = 8
_CHUNK = _B_PER_W // _N_CHUNKS


def _build_gather():
    mesh = plsc.VectorSubcoreMesh(core_axis_name="c", subcore_axis_name="s")

    @functools.partial(
        pl.kernel,
        out_type=jax.ShapeDtypeStruct((BATCH, EMB_DIM), jnp.float32),
        mesh=mesh,
        scratch_types=[
            pltpu.VMEM((_B_PER_W,), jnp.int32),
            pltpu.VMEM((_B_PER_W, EMB_DIM), jnp.float32),
            pltpu.VMEM_SHARED((TIMESTEPS, EMB_DIM), jnp.float32),
            pltpu.SemaphoreType.DMA((_N_CHUNKS,)),
            pltpu.SemaphoreType.DMA,
            pltpu.SemaphoreType.DMA,
        ],
    )
    def gather_kernel(table_hbm, idx_hbm, out_hbm, idx_v, rows_v, table_sp, gsems, ssem, isem):
        sid = lax.axis_index("s")
        wid = sid * _NUM_CORES + lax.axis_index("c")
        base = wid * _B_PER_W
        rows_lo = sid * 64
        n_rows = jnp.where(sid == _NUM_SUBCORES - 1, TIMESTEPS - 64 * (_NUM_SUBCORES - 1), 64)
        idx_cp = pltpu.async_copy(idx_hbm.at[pl.ds(base, _B_PER_W)], idx_v, isem)
        stage_cp = pltpu.async_copy(
            table_hbm.at[pl.ds(rows_lo, n_rows)],
            table_sp.at[pl.ds(rows_lo, n_rows)],
            ssem)
        stage_cp.wait()
        idx_cp.wait()
        plsc.subcore_barrier()
        gathers = []
        for c in range(_N_CHUNKS):
            lo = c * _CHUNK
            gathers.append(pltpu.async_copy(
                table_sp.at[idx_v.at[pl.ds(lo, _CHUNK)]],
                rows_v.at[pl.ds(lo, _CHUNK)],
                gsems.at[c]))
        scatters = []
        for c in range(_N_CHUNKS):
            lo = c * _CHUNK
            gathers[c].wait()
            scatters.append(pltpu.async_copy(
                rows_v.at[pl.ds(lo, _CHUNK)],
                out_hbm.at[pl.ds(base + lo, _CHUNK)],
                ssem))
        for s in scatters:
            s.wait()

    return gather_kernel


_gather = _build_gather()


@jax.jit
def kernel(timestep, embedding):
    return _gather(embedding, timestep)


# speedup vs baseline: 1.0167x; 1.0107x over previous
import functools

import jax
import jax.numpy as jnp
from jax import lax
from jax.experimental import pallas as pl
from jax.experimental.pallas import tpu as pltpu, tpu_sc as plsc

EMB_DIM = 128
TIMESTEPS = 1000
BATCH = 16384

_NUM_CORES = 2
_NUM_SUBCORES = 16
_NUM_WORKERS = _NUM_CORES * _NUM_SUBCORES
_B_PER_W = BATCH // _NUM_WORKERS
_N_CHUNKS = 8
_CHUNK = _B_PER_W // _N_CHUNKS


def _build_gather():
    mesh = plsc.VectorSubcoreMesh(core_axis_name="c", subcore_axis_name="s")

    @functools.partial(
        pl.kernel,
        out_type=jax.ShapeDtypeStruct((BATCH, EMB_DIM), jnp.float32),
        mesh=mesh,
        scratch_types=[
            pltpu.VMEM((_B_PER_W,), jnp.int32),
            pltpu.VMEM((_B_PER_W, EMB_DIM), jnp.float32),
            pltpu.VMEM_SHARED((TIMESTEPS, EMB_DIM), jnp.float32),
            pltpu.SemaphoreType.DMA((_N_CHUNKS,)),
            pltpu.SemaphoreType.DMA,
            pltpu.SemaphoreType.DMA,
        ],
    )
    def gather_kernel(table_hbm, idx_hbm, out_hbm, idx_v, rows_v, table_sp, gsems, ssem, isem):
        sid = lax.axis_index("s")
        wid = sid * _NUM_CORES + lax.axis_index("c")
        base = wid * _B_PER_W
        rows_lo = sid * 64
        n_rows = jnp.where(sid == _NUM_SUBCORES - 1, TIMESTEPS - 64 * (_NUM_SUBCORES - 1), 64)
        idx_cp = pltpu.async_copy(idx_hbm.at[pl.ds(base, _B_PER_W)], idx_v, isem)
        stage_cp = pltpu.async_copy(
            table_hbm.at[pl.ds(rows_lo, n_rows)],
            table_sp.at[pl.ds(rows_lo, n_rows)],
            ssem)
        idx_cp.wait()
        gathers = [pltpu.async_copy(
            table_hbm.at[idx_v.at[pl.ds(0, _CHUNK)]],
            rows_v.at[pl.ds(0, _CHUNK)],
            gsems.at[0])]
        stage_cp.wait()
        plsc.subcore_barrier()
        for c in range(1, _N_CHUNKS):
            lo = c * _CHUNK
            gathers.append(pltpu.async_copy(
                table_sp.at[idx_v.at[pl.ds(lo, _CHUNK)]],
                rows_v.at[pl.ds(lo, _CHUNK)],
                gsems.at[c]))
        scatters = []
        for c in range(_N_CHUNKS):
            lo = c * _CHUNK
            gathers[c].wait()
            scatters.append(pltpu.async_copy(
                rows_v.at[pl.ds(lo, _CHUNK)],
                out_hbm.at[pl.ds(base + lo, _CHUNK)],
                ssem))
        for s in scatters:
            s.wait()

    return gather_kernel


_gather = _build_gather()


@jax.jit
def kernel(timestep, embedding):
    return _gather(embedding, timestep)
